# Initial kernel scaffold; baseline (speedup 1.0000x reference)
import jax
import jax.numpy as jnp
from jax.experimental import pallas as pl


def kernel(data, adj1, adj2, W_enc, b_enc, W_f1, b_f1, W_f2, b_f2, W_dec, b_dec):
    raise NotImplementedError("write your pallas kernel here")


# speedup vs baseline: 4.7904x; 4.7904x over previous
import functools

import jax
import jax.numpy as jnp
from jax import lax
from jax.experimental import pallas as pl
from jax.experimental.pallas import tpu as pltpu
from jax.experimental.pallas import tpu_sc as plsc

N = 10000
E = 320000
D_IN = 128
D_H = 256
D_Z = 64

NUM_CORES = 2
NUM_TILES = 16
EPT = E // NUM_TILES
CHUNK = 80
NCHUNK = EPT // CHUNK
N_PAD = 10240
ROWS_PT = N_PAD // NUM_TILES
FLUSH_B = 80


def _sc_aggregate(x, srcs, dsts, zrow, ones):
  mesh = plsc.VectorSubcoreMesh(core_axis_name="c", subcore_axis_name="s",
                                num_cores=NUM_CORES)

  @functools.partial(
      pl.kernel,
      out_type=(
          jax.ShapeDtypeStruct((NUM_CORES, N_PAD, D_IN), jnp.float32),
          jax.ShapeDtypeStruct((NUM_CORES, N_PAD, D_IN), jnp.float32),
      ),
      mesh=mesh,
      scratch_types=dict(
          idx_s=pltpu.VMEM((CHUNK,), jnp.int32),
          idx_d=pltpu.VMEM((CHUNK,), jnp.int32),
          rows=pltpu.VMEM((CHUNK, D_IN), jnp.float32),
          sem=pltpu.SemaphoreType.DMA,
          acc=pltpu.VMEM_SHARED((N_PAD, D_IN), jnp.float32),
      ),
  )
  def k(x_hbm, srcs_hbm, dsts_hbm, zrow_hbm, ones_hbm, sums_hbm, degs_hbm,
        idx_s, idx_d, rows, sem, acc):
    c = lax.axis_index("c")
    s = lax.axis_index("s")
    r0 = s * ROWS_PT
    base = c * E + s * EPT

    def zero_acc():
      pltpu.sync_copy(zrow_hbm, rows)
      for j in range(ROWS_PT // FLUSH_B):
        pltpu.sync_copy(rows, acc.at[pl.ds(r0 + j * FLUSH_B, FLUSH_B)])

    def flush(out_hbm):
      for j in range(ROWS_PT // FLUSH_B):
        r = r0 + j * FLUSH_B
        pltpu.sync_copy(acc.at[pl.ds(r, FLUSH_B)], rows)
        pltpu.sync_copy(rows, out_hbm.at[c, pl.ds(r, FLUSH_B)])

    zero_acc()
    plsc.subcore_barrier()

    @pl.loop(0, NCHUNK)
    def step(i):
      off = pl.multiple_of(base + i * CHUNK, 8)
      pltpu.sync_copy(srcs_hbm.at[pl.ds(off, CHUNK)], idx_s)
      pltpu.sync_copy(dsts_hbm.at[pl.ds(off, CHUNK)], idx_d)
      pltpu.async_copy(x_hbm.at[idx_s], rows, sem).wait()
      pltpu.sync_copy(rows, acc.at[idx_d], add=True)

    plsc.subcore_barrier()
    flush(sums_hbm)
    plsc.subcore_barrier()

    zero_acc()
    pltpu.sync_copy(ones_hbm, rows)
    plsc.subcore_barrier()

    @pl.loop(0, NCHUNK)
    def step2(i):
      off = pl.multiple_of(base + i * CHUNK, 8)
      pltpu.sync_copy(dsts_hbm.at[pl.ds(off, CHUNK)], idx_d)
      pltpu.sync_copy(rows, acc.at[idx_d], add=True)

    plsc.subcore_barrier()
    flush(degs_hbm)

  return k(x, srcs, dsts, zrow, ones)


BN = 400


def _dense_body(S_ref, D_ref, Wenc_ref, benc_ref, Wf1_ref, bf1_ref,
                Wf2_ref, bf2_ref, Wdec_ref, bdec_ref, xrec_ref, zpre_ref):
  S = S_ref[...]
  D = D_ref[...]
  Wenc = Wenc_ref[...]
  benc = benc_ref[...]

  def view(v):
    d = D[v, :, 0:1]
    m = S[v] / jnp.maximum(d, 1.0)
    pre = jnp.dot(m, Wenc, preferred_element_type=jnp.float32)
    pre = pre + benc[None, :] * (d > 0).astype(jnp.float32)
    return jnp.where(pre > 0, pre, jnp.exp(pre) - 1.0)

  z1 = view(0)
  z2 = view(1)
  Wf1 = Wf1_ref[...]
  h = (jnp.dot(z1, Wf1[:D_H], preferred_element_type=jnp.float32)
       + jnp.dot(z2, Wf1[D_H:], preferred_element_type=jnp.float32)
       + bf1_ref[...][None, :])
  h = jnp.maximum(h, 0.0)
  zp = jnp.dot(h, Wf2_ref[...], preferred_element_type=jnp.float32)
  zp = zp + bf2_ref[...][None, :]
  xr = jnp.dot(zp, Wdec_ref[...], preferred_element_type=jnp.float32)
  xr = xr + bdec_ref[...][None, :]
  xrec_ref[...] = xr
  zpre_ref[...] = zp


def _dense(sums, degs, W_enc, b_enc, W_f1, b_f1, W_f2, b_f2, W_dec, b_dec):
  grid = (N // BN,)
  full = lambda shape: pl.BlockSpec(shape, lambda i: (0,) * len(shape))
  return pl.pallas_call(
      _dense_body,
      grid=grid,
      in_specs=[
          pl.BlockSpec((NUM_CORES, BN, D_IN), lambda i: (0, i, 0)),
          pl.BlockSpec((NUM_CORES, BN, D_IN), lambda i: (0, i, 0)),
          full((D_IN, D_H)),
          full((D_H,)),
          full((2 * D_H, 128)),
          full((128,)),
          full((128, D_Z)),
          full((D_Z,)),
          full((D_Z, D_IN)),
          full((D_IN,)),
      ],
      out_specs=[
          pl.BlockSpec((BN, D_IN), lambda i: (i, 0)),
          pl.BlockSpec((BN, D_Z), lambda i: (i, 0)),
      ],
      out_shape=[
          jax.ShapeDtypeStruct((N, D_IN), jnp.float32),
          jax.ShapeDtypeStruct((N, D_Z), jnp.float32),
      ],
  )(sums, degs, W_enc, b_enc, W_f1, b_f1, W_f2, b_f2, W_dec, b_dec)


def kernel(data, adj1, adj2, W_enc, b_enc, W_f1, b_f1, W_f2, b_f2, W_dec, b_dec):
  srcs = jnp.concatenate([adj1[0], adj2[0]])
  dsts = jnp.concatenate([adj1[1], adj2[1]])
  zrow = jnp.zeros((FLUSH_B, D_IN), jnp.float32)
  ones = jnp.ones((CHUNK, D_IN), jnp.float32)
  sums, degs = _sc_aggregate(data, srcs, dsts, zrow, ones)
  x_rec, z_pretrain = _dense(sums, degs, W_enc, b_enc, W_f1, b_f1,
                             W_f2, b_f2, W_dec, b_dec)
  return (x_rec, z_pretrain)


# speedup vs baseline: 7.5359x; 1.5731x over previous
import functools

import jax
import jax.numpy as jnp
from jax import lax
from jax.experimental import pallas as pl
from jax.experimental.pallas import tpu as pltpu
from jax.experimental.pallas import tpu_sc as plsc

N = 10000
E = 320000
D_IN = 128
D_H = 256
D_Z = 64

NUM_CORES = 2
NUM_TILES = 16
EPT = E // NUM_TILES
CHUNK = 80
NCHUNK = EPT // CHUNK
N_PAD = 10240
ROWS_PT = N_PAD // NUM_TILES
FLUSH_B = 80


def _sc_aggregate(x, srcs, dsts, zrow, ones):
  mesh = plsc.VectorSubcoreMesh(core_axis_name="c", subcore_axis_name="s",
                                num_cores=NUM_CORES)

  @functools.partial(
      pl.kernel,
      out_type=(
          jax.ShapeDtypeStruct((NUM_CORES, N_PAD, D_IN), jnp.float32),
          jax.ShapeDtypeStruct((NUM_CORES, N_PAD, D_IN), jnp.float32),
      ),
      mesh=mesh,
      scratch_types=dict(
          idx_s0=pltpu.VMEM((CHUNK,), jnp.int32),
          idx_s1=pltpu.VMEM((CHUNK,), jnp.int32),
          idx_d0=pltpu.VMEM((CHUNK,), jnp.int32),
          idx_d1=pltpu.VMEM((CHUNK,), jnp.int32),
          rows0=pltpu.VMEM((CHUNK, D_IN), jnp.float32),
          rows1=pltpu.VMEM((CHUNK, D_IN), jnp.float32),
          semg0=pltpu.SemaphoreType.DMA,
          semg1=pltpu.SemaphoreType.DMA,
          sems0=pltpu.SemaphoreType.DMA,
          sems1=pltpu.SemaphoreType.DMA,
          acc=pltpu.VMEM_SHARED((N_PAD, D_IN), jnp.float32),
      ),
  )
  def k(x_hbm, srcs_hbm, dsts_hbm, zrow_hbm, ones_hbm, sums_hbm, degs_hbm,
        idx_s0, idx_s1, idx_d0, idx_d1, rows0, rows1,
        semg0, semg1, sems0, sems1, acc):
    c = lax.axis_index("c")
    s = lax.axis_index("s")
    r0 = s * ROWS_PT
    base = c * E + s * EPT
    NPAIR = NCHUNK // 2

    def off(j):
      return pl.multiple_of(base + j * CHUNK, 8)

    def zero_acc():
      pltpu.sync_copy(zrow_hbm, rows0)
      for j in range(ROWS_PT // FLUSH_B):
        pltpu.sync_copy(rows0, acc.at[pl.ds(r0 + j * FLUSH_B, FLUSH_B)])

    def flush(out_hbm):
      for j in range(ROWS_PT // FLUSH_B):
        r = r0 + j * FLUSH_B
        pltpu.sync_copy(acc.at[pl.ds(r, FLUSH_B)], rows0)
        pltpu.sync_copy(rows0, out_hbm.at[c, pl.ds(r, FLUSH_B)])

    zero_acc()
    plsc.subcore_barrier()

    pltpu.sync_copy(srcs_hbm.at[pl.ds(off(0), CHUNK)], idx_s0)
    pltpu.sync_copy(dsts_hbm.at[pl.ds(off(0), CHUNK)], idx_d0)
    pltpu.async_copy(x_hbm.at[idx_s0], rows0, semg0)
    pltpu.sync_copy(srcs_hbm.at[pl.ds(off(1), CHUNK)], idx_s1)
    pltpu.sync_copy(dsts_hbm.at[pl.ds(off(1), CHUNK)], idx_d1)

    @pl.loop(0, NPAIR)
    def pair(p):
      a = 2 * p
      gb = pltpu.async_copy(x_hbm.at[idx_s1], rows1, semg1)
      pltpu.make_async_copy(x_hbm.at[idx_s0], rows0, semg0).wait()
      sa = pltpu.async_copy(rows0, acc.at[idx_d0], sems0, add=True)
      gb.wait()
      sb = pltpu.async_copy(rows1, acc.at[idx_d1], sems1, add=True)
      sa.wait()

      @pl.when(p < NPAIR - 1)
      def _():
        pltpu.sync_copy(srcs_hbm.at[pl.ds(off(a + 2), CHUNK)], idx_s0)
        pltpu.sync_copy(dsts_hbm.at[pl.ds(off(a + 2), CHUNK)], idx_d0)
        pltpu.async_copy(x_hbm.at[idx_s0], rows0, semg0)

      sb.wait()

      @pl.when(p < NPAIR - 1)
      def _():
        pltpu.sync_copy(srcs_hbm.at[pl.ds(off(a + 3), CHUNK)], idx_s1)
        pltpu.sync_copy(dsts_hbm.at[pl.ds(off(a + 3), CHUNK)], idx_d1)

    plsc.subcore_barrier()
    flush(sums_hbm)
    plsc.subcore_barrier()

    zero_acc()
    pltpu.sync_copy(ones_hbm, rows0)
    plsc.subcore_barrier()

    pltpu.sync_copy(dsts_hbm.at[pl.ds(off(0), CHUNK)], idx_d0)

    @pl.loop(0, NPAIR)
    def pair2(p):
      a = 2 * p
      sa = pltpu.async_copy(rows0, acc.at[idx_d0], sems0, add=True)
      pltpu.sync_copy(dsts_hbm.at[pl.ds(off(a + 1), CHUNK)], idx_d1)
      sb = pltpu.async_copy(rows0, acc.at[idx_d1], sems1, add=True)
      sa.wait()

      @pl.when(p < NPAIR - 1)
      def _():
        pltpu.sync_copy(dsts_hbm.at[pl.ds(off(a + 2), CHUNK)], idx_d0)

      sb.wait()

    plsc.subcore_barrier()
    flush(degs_hbm)

  return k(x, srcs, dsts, zrow, ones)


BN = 400


def _dense_body(S_ref, D_ref, Wenc_ref, benc_ref, Wf1_ref, bf1_ref,
                Wf2_ref, bf2_ref, Wdec_ref, bdec_ref, xrec_ref, zpre_ref):
  S = S_ref[...]
  D = D_ref[...]
  Wenc = Wenc_ref[...]
  benc = benc_ref[...]

  def view(v):
    d = D[v, :, 0:1]
    m = S[v] / jnp.maximum(d, 1.0)
    pre = jnp.dot(m, Wenc, preferred_element_type=jnp.float32)
    pre = pre + benc[None, :] * (d > 0).astype(jnp.float32)
    return jnp.where(pre > 0, pre, jnp.exp(pre) - 1.0)

  z1 = view(0)
  z2 = view(1)
  Wf1 = Wf1_ref[...]
  h = (jnp.dot(z1, Wf1[:D_H], preferred_element_type=jnp.float32)
       + jnp.dot(z2, Wf1[D_H:], preferred_element_type=jnp.float32)
       + bf1_ref[...][None, :])
  h = jnp.maximum(h, 0.0)
  zp = jnp.dot(h, Wf2_ref[...], preferred_element_type=jnp.float32)
  zp = zp + bf2_ref[...][None, :]
  xr = jnp.dot(zp, Wdec_ref[...], preferred_element_type=jnp.float32)
  xr = xr + bdec_ref[...][None, :]
  xrec_ref[...] = xr
  zpre_ref[...] = zp


def _dense(sums, degs, W_enc, b_enc, W_f1, b_f1, W_f2, b_f2, W_dec, b_dec):
  grid = (N // BN,)
  full = lambda shape: pl.BlockSpec(shape, lambda i: (0,) * len(shape))
  return pl.pallas_call(
      _dense_body,
      grid=grid,
      in_specs=[
          pl.BlockSpec((NUM_CORES, BN, D_IN), lambda i: (0, i, 0)),
          pl.BlockSpec((NUM_CORES, BN, D_IN), lambda i: (0, i, 0)),
          full((D_IN, D_H)),
          full((D_H,)),
          full((2 * D_H, 128)),
          full((128,)),
          full((128, D_Z)),
          full((D_Z,)),
          full((D_Z, D_IN)),
          full((D_IN,)),
      ],
      out_specs=[
          pl.BlockSpec((BN, D_IN), lambda i: (i, 0)),
          pl.BlockSpec((BN, D_Z), lambda i: (i, 0)),
      ],
      out_shape=[
          jax.ShapeDtypeStruct((N, D_IN), jnp.float32),
          jax.ShapeDtypeStruct((N, D_Z), jnp.float32),
      ],
  )(sums, degs, W_enc, b_enc, W_f1, b_f1, W_f2, b_f2, W_dec, b_dec)


def kernel(data, adj1, adj2, W_enc, b_enc, W_f1, b_f1, W_f2, b_f2, W_dec, b_dec):
  srcs = jnp.concatenate([adj1[0], adj2[0]])
  dsts = jnp.concatenate([adj1[1], adj2[1]])
  zrow = jnp.zeros((FLUSH_B, D_IN), jnp.float32)
  ones = jnp.ones((CHUNK, D_IN), jnp.float32)
  sums, degs = _sc_aggregate(data, srcs, dsts, zrow, ones)
  x_rec, z_pretrain = _dense(sums, degs, W_enc, b_enc, W_f1, b_f1,
                             W_f2, b_f2, W_dec, b_dec)
  return (x_rec, z_pretrain)


# speedup vs baseline: 8.9696x; 1.1903x over previous
import functools

import jax
import jax.numpy as jnp
from jax import lax
from jax.experimental import pallas as pl
from jax.experimental.pallas import tpu as pltpu
from jax.experimental.pallas import tpu_sc as plsc

N = 10000
E = 320000
D_IN = 128
D_H = 256
D_Z = 64

NUM_CORES = 2
NUM_TILES = 16
EPT = E // NUM_TILES
CHUNK = 128
NMAIN = EPT // CHUNK
TAIL = EPT - NMAIN * CHUNK
N_PAD = 10240
ROWS_PT = N_PAD // NUM_TILES
FLUSH_B = 128


def _sc_aggregate(x, srcs, dsts, zrow, ones):
  mesh = plsc.VectorSubcoreMesh(core_axis_name="c", subcore_axis_name="s",
                                num_cores=NUM_CORES)

  @functools.partial(
      pl.kernel,
      out_type=(
          jax.ShapeDtypeStruct((NUM_CORES, N_PAD, D_IN), jnp.float32),
          jax.ShapeDtypeStruct((NUM_CORES, N_PAD, D_IN), jnp.float32),
      ),
      mesh=mesh,
      scratch_types=dict(
          idx_s0=pltpu.VMEM((CHUNK,), jnp.int32),
          idx_s1=pltpu.VMEM((CHUNK,), jnp.int32),
          idx_d0=pltpu.VMEM((CHUNK,), jnp.int32),
          idx_d1=pltpu.VMEM((CHUNK,), jnp.int32),
          idx_st=pltpu.VMEM((TAIL,), jnp.int32),
          idx_dt=pltpu.VMEM((TAIL,), jnp.int32),
          rows0=pltpu.VMEM((CHUNK, D_IN), jnp.float32),
          rows1=pltpu.VMEM((CHUNK, D_IN), jnp.float32),
          rows_t=pltpu.VMEM((TAIL, D_IN), jnp.float32),
          semg0=pltpu.SemaphoreType.DMA,
          semg1=pltpu.SemaphoreType.DMA,
          sems0=pltpu.SemaphoreType.DMA,
          sems1=pltpu.SemaphoreType.DMA,
          acc=pltpu.VMEM_SHARED((N_PAD, D_IN), jnp.float32),
      ),
  )
  def k(x_hbm, srcs_hbm, dsts_hbm, zrow_hbm, ones_hbm, sums_hbm, degs_hbm,
        idx_s0, idx_s1, idx_d0, idx_d1, idx_st, idx_dt, rows0, rows1,
        rows_t, semg0, semg1, sems0, sems1, acc):
    c = lax.axis_index("c")
    s = lax.axis_index("s")
    r0 = s * ROWS_PT
    base = c * E + s * EPT
    NPAIR = NMAIN // 2
    toff = pl.multiple_of(base + NMAIN * CHUNK, 8)

    def off(j):
      return pl.multiple_of(base + j * CHUNK, 8)

    def zero_acc():
      pltpu.sync_copy(zrow_hbm, rows0)
      for j in range(ROWS_PT // FLUSH_B):
        pltpu.sync_copy(rows0, acc.at[pl.ds(r0 + j * FLUSH_B, FLUSH_B)])

    def flush(out_hbm):
      for j in range(ROWS_PT // FLUSH_B):
        r = r0 + j * FLUSH_B
        pltpu.sync_copy(acc.at[pl.ds(r, FLUSH_B)], rows0)
        pltpu.sync_copy(rows0, out_hbm.at[c, pl.ds(r, FLUSH_B)])

    zero_acc()
    plsc.subcore_barrier()

    pltpu.sync_copy(srcs_hbm.at[pl.ds(off(0), CHUNK)], idx_s0)
    pltpu.sync_copy(dsts_hbm.at[pl.ds(off(0), CHUNK)], idx_d0)
    pltpu.async_copy(x_hbm.at[idx_s0], rows0, semg0)
    pltpu.sync_copy(srcs_hbm.at[pl.ds(off(1), CHUNK)], idx_s1)
    pltpu.sync_copy(dsts_hbm.at[pl.ds(off(1), CHUNK)], idx_d1)

    @pl.loop(0, NPAIR)
    def pair(p):
      a = 2 * p
      gb = pltpu.async_copy(x_hbm.at[idx_s1], rows1, semg1)
      pltpu.make_async_copy(x_hbm.at[idx_s0], rows0, semg0).wait()
      sa = pltpu.async_copy(rows0, acc.at[idx_d0], sems0, add=True)
      gb.wait()
      sb = pltpu.async_copy(rows1, acc.at[idx_d1], sems1, add=True)
      sa.wait()

      @pl.when(p < NPAIR - 1)
      def _():
        pltpu.sync_copy(srcs_hbm.at[pl.ds(off(a + 2), CHUNK)], idx_s0)
        pltpu.sync_copy(dsts_hbm.at[pl.ds(off(a + 2), CHUNK)], idx_d0)
        pltpu.async_copy(x_hbm.at[idx_s0], rows0, semg0)

      sb.wait()

      @pl.when(p < NPAIR - 1)
      def _():
        pltpu.sync_copy(srcs_hbm.at[pl.ds(off(a + 3), CHUNK)], idx_s1)
        pltpu.sync_copy(dsts_hbm.at[pl.ds(off(a + 3), CHUNK)], idx_d1)

    pltpu.sync_copy(srcs_hbm.at[pl.ds(toff, TAIL)], idx_st)
    pltpu.sync_copy(dsts_hbm.at[pl.ds(toff, TAIL)], idx_dt)
    pltpu.async_copy(x_hbm.at[idx_st], rows_t, semg0).wait()
    pltpu.sync_copy(rows_t, acc.at[idx_dt], add=True)

    plsc.subcore_barrier()
    flush(sums_hbm)
    plsc.subcore_barrier()

    zero_acc()
    pltpu.sync_copy(ones_hbm, rows0)
    plsc.subcore_barrier()

    pltpu.sync_copy(dsts_hbm.at[pl.ds(off(0), CHUNK)], idx_d0)

    @pl.loop(0, NPAIR)
    def pair2(p):
      a = 2 * p
      sa = pltpu.async_copy(rows0, acc.at[idx_d0], sems0, add=True)
      pltpu.sync_copy(dsts_hbm.at[pl.ds(off(a + 1), CHUNK)], idx_d1)
      sb = pltpu.async_copy(rows0, acc.at[idx_d1], sems1, add=True)
      sa.wait()

      @pl.when(p < NPAIR - 1)
      def _():
        pltpu.sync_copy(dsts_hbm.at[pl.ds(off(a + 2), CHUNK)], idx_d0)

      sb.wait()

    pltpu.sync_copy(dsts_hbm.at[pl.ds(toff, TAIL)], idx_dt)
    pltpu.sync_copy(ones_hbm.at[pl.ds(0, TAIL)], rows_t)
    pltpu.sync_copy(rows_t, acc.at[idx_dt], add=True)

    plsc.subcore_barrier()
    flush(degs_hbm)

  return k(x, srcs, dsts, zrow, ones)


BN = 400


def _dense_body(S_ref, D_ref, Wenc_ref, benc_ref, Wf1_ref, bf1_ref,
                Wf2_ref, bf2_ref, Wdec_ref, bdec_ref, xrec_ref, zpre_ref):
  S = S_ref[...]
  D = D_ref[...]
  Wenc = Wenc_ref[...]
  benc = benc_ref[...]

  def view(v):
    d = D[v, :, 0:1]
    m = S[v] / jnp.maximum(d, 1.0)
    pre = jnp.dot(m, Wenc, preferred_element_type=jnp.float32)
    pre = pre + benc[None, :] * (d > 0).astype(jnp.float32)
    return jnp.where(pre > 0, pre, jnp.exp(pre) - 1.0)

  z1 = view(0)
  z2 = view(1)
  Wf1 = Wf1_ref[...]
  h = (jnp.dot(z1, Wf1[:D_H], preferred_element_type=jnp.float32)
       + jnp.dot(z2, Wf1[D_H:], preferred_element_type=jnp.float32)
       + bf1_ref[...][None, :])
  h = jnp.maximum(h, 0.0)
  zp = jnp.dot(h, Wf2_ref[...], preferred_element_type=jnp.float32)
  zp = zp + bf2_ref[...][None, :]
  xr = jnp.dot(zp, Wdec_ref[...], preferred_element_type=jnp.float32)
  xr = xr + bdec_ref[...][None, :]
  xrec_ref[...] = xr
  zpre_ref[...] = zp


def _dense(sums, degs, W_enc, b_enc, W_f1, b_f1, W_f2, b_f2, W_dec, b_dec):
  grid = (N // BN,)
  full = lambda shape: pl.BlockSpec(shape, lambda i: (0,) * len(shape))
  return pl.pallas_call(
      _dense_body,
      grid=grid,
      in_specs=[
          pl.BlockSpec((NUM_CORES, BN, D_IN), lambda i: (0, i, 0)),
          pl.BlockSpec((NUM_CORES, BN, D_IN), lambda i: (0, i, 0)),
          full((D_IN, D_H)),
          full((D_H,)),
          full((2 * D_H, 128)),
          full((128,)),
          full((128, D_Z)),
          full((D_Z,)),
          full((D_Z, D_IN)),
          full((D_IN,)),
      ],
      out_specs=[
          pl.BlockSpec((BN, D_IN), lambda i: (i, 0)),
          pl.BlockSpec((BN, D_Z), lambda i: (i, 0)),
      ],
      out_shape=[
          jax.ShapeDtypeStruct((N, D_IN), jnp.float32),
          jax.ShapeDtypeStruct((N, D_Z), jnp.float32),
      ],
  )(sums, degs, W_enc, b_enc, W_f1, b_f1, W_f2, b_f2, W_dec, b_dec)


def kernel(data, adj1, adj2, W_enc, b_enc, W_f1, b_f1, W_f2, b_f2, W_dec, b_dec):
  srcs = jnp.concatenate([adj1[0], adj2[0]])
  dsts = jnp.concatenate([adj1[1], adj2[1]])
  zrow = jnp.zeros((FLUSH_B, D_IN), jnp.float32)
  ones = jnp.ones((CHUNK, D_IN), jnp.float32)
  sums, degs = _sc_aggregate(data, srcs, dsts, zrow, ones)
  x_rec, z_pretrain = _dense(sums, degs, W_enc, b_enc, W_f1, b_f1,
                             W_f2, b_f2, W_dec, b_dec)
  return (x_rec, z_pretrain)
